# baseline (speedup 1.0000x reference)
import functools

import jax
import jax.numpy as jnp
from jax import lax
from jax.experimental import pallas as pl
from jax.experimental.pallas import tpu as pltpu
from jax.experimental.pallas import tpu_sc as plsc

VOCAB = 1000000
HIDDEN = 128
B, L = 1024, 200
CHUNK = 40
NW = 32
NSLICES = 8
BSL = B // NSLICES
TSL = BSL * L
CPWS = TSL // (CHUNK * NW)
TPWS = CPWS * CHUNK
NBUF = 4
BB = 32


def _sc_body(ids_hbm, word_hbm, raw_hbm, ids_v, rows, gsems, osems):
    wid = lax.axis_index("s") * 2 + lax.axis_index("c")
    tok0 = wid * TPWS

    pltpu.sync_copy(ids_hbm.at[pl.ds(tok0, TPWS)], ids_v)

    for b in range(NBUF - 1):
        pltpu.async_copy(
            word_hbm.at[ids_v.at[pl.ds(b * CHUNK, CHUNK)]], rows[b],
            gsems[b])

    def step(c, b):
        b2 = (b + NBUF - 1) % NBUF

        @pl.when(c + NBUF - 1 < CPWS)
        def _():
            @pl.when(c > 0)
            def _():
                pltpu.make_async_copy(
                    rows[b2], raw_hbm.at[pl.ds(tok0, CHUNK)],
                    osems[b2]).wait()

            nbase = pl.multiple_of((c + NBUF - 1) * CHUNK, CHUNK)
            pltpu.async_copy(
                word_hbm.at[ids_v.at[pl.ds(nbase, CHUNK)]], rows[b2],
                gsems[b2])

        pltpu.make_async_copy(
            word_hbm.at[pl.ds(0, CHUNK)], rows[b], gsems[b]).wait()
        cbase = pl.multiple_of(c * CHUNK, CHUNK)
        pltpu.async_copy(rows[b], raw_hbm.at[pl.ds(tok0 + cbase, CHUNK)],
                         osems[b])

    def ring(i, _):
        c0 = i * NBUF
        for b in range(NBUF):
            step(c0 + b, b)
        return 0

    lax.fori_loop(0, CPWS // NBUF, ring, 0)

    for b in range(NBUF):
        pltpu.make_async_copy(
            rows[b], raw_hbm.at[pl.ds(tok0, CHUNK)], osems[b]).wait()


def _sc_gather(ids_slice, word_emb):
    mesh = plsc.VectorSubcoreMesh(core_axis_name="c", subcore_axis_name="s")

    def body(ids_hbm, word_hbm, raw_hbm, *scr):
        _sc_body(ids_hbm, word_hbm, raw_hbm, scr[0],
                 list(scr[1:1 + NBUF]), list(scr[1 + NBUF:1 + 2 * NBUF]),
                 list(scr[1 + 2 * NBUF:]))

    f = functools.partial(
        pl.kernel,
        out_type=jax.ShapeDtypeStruct((TSL, HIDDEN), jnp.float32),
        mesh=mesh,
        scratch_types=(
            [pltpu.VMEM((TPWS,), jnp.int32)]
            + [pltpu.VMEM((CHUNK, HIDDEN), jnp.float32)] * NBUF
            + [pltpu.SemaphoreType.DMA] * (2 * NBUF)
        ),
    )(body)
    return f(ids_slice, word_emb)


def _tc_body(raw_ref, tt_ref, pos_ref, typ_ref, gamma_ref, beta_ref,
             out_ref):
    x = raw_ref[...]
    ttf = tt_ref[...].astype(jnp.float32)
    pos = pos_ref[...]
    typ0 = typ_ref[0]
    typd = typ_ref[1] - typ0
    x = x + pos[None] + typ0[None, None] + ttf[..., None] * typd[None, None]
    mean = jnp.mean(x, axis=-1, keepdims=True)
    var = jnp.mean(x * x, axis=-1, keepdims=True) - mean * mean
    y = (x - mean) * lax.rsqrt(var + 1e-12)
    out_ref[...] = y * gamma_ref[...][None, None] + beta_ref[...][None, None]


def _tc_ln(raw_slice, tt_slice, pos_emb, type_emb, gamma, beta):
    grid = (BSL // BB,)
    return pl.pallas_call(
        _tc_body,
        grid=grid,
        in_specs=[
            pl.BlockSpec((BB, L, HIDDEN), lambda i: (i, 0, 0)),
            pl.BlockSpec((BB, L), lambda i: (i, 0)),
            pl.BlockSpec((L, HIDDEN), lambda i: (0, 0)),
            pl.BlockSpec((2, HIDDEN), lambda i: (0, 0)),
            pl.BlockSpec((HIDDEN,), lambda i: (0,)),
            pl.BlockSpec((HIDDEN,), lambda i: (0,)),
        ],
        out_specs=pl.BlockSpec((BB, L, HIDDEN), lambda i: (i, 0, 0)),
        out_shape=jax.ShapeDtypeStruct((BSL, L, HIDDEN), jnp.float32),
    )(raw_slice, tt_slice, pos_emb[:L], type_emb, gamma, beta)


@jax.jit
def _embed(input_ids, token_type_ids, word_emb, pos_emb, type_emb, gamma,
           beta):
    ids1 = input_ids.reshape(B * L)
    raws = [
        _sc_gather(lax.dynamic_slice_in_dim(ids1, s * TSL, TSL), word_emb)
        for s in range(NSLICES)
    ]
    outs = [
        _tc_ln(raws[s].reshape(BSL, L, HIDDEN),
               lax.dynamic_slice_in_dim(token_type_ids, s * BSL, BSL),
               pos_emb, type_emb, gamma, beta)
        for s in range(NSLICES)
    ]
    return jnp.concatenate(outs, axis=0)


def kernel(input_ids, token_type_ids, word_emb, pos_emb, type_emb, gamma,
           beta):
    return _embed(input_ids, token_type_ids, word_emb, pos_emb, type_emb,
                  gamma, beta)


# speedup vs baseline: 1.1048x; 1.1048x over previous
import functools

import jax
import jax.numpy as jnp
from jax import lax
from jax.experimental import pallas as pl
from jax.experimental.pallas import tpu as pltpu
from jax.experimental.pallas import tpu_sc as plsc

VOCAB = 1000000
HIDDEN = 128
B, L = 1024, 200
CHUNK = 40
NW = 32
NSLICES = 4
BSL = B // NSLICES
TSL = BSL * L
CPWS = TSL // (CHUNK * NW)
TPWS = CPWS * CHUNK
NBUF = 4
BB = 32


def _sc_body(ids_hbm, word_hbm, raw_hbm, ids_v, rows, gsems, osems):
    wid = lax.axis_index("s") * 2 + lax.axis_index("c")
    tok0 = wid * TPWS

    pltpu.sync_copy(ids_hbm.at[pl.ds(tok0, TPWS)], ids_v)

    for b in range(NBUF - 1):
        pltpu.async_copy(
            word_hbm.at[ids_v.at[pl.ds(b * CHUNK, CHUNK)]], rows[b],
            gsems[b])

    def step(c, b):
        b2 = (b + NBUF - 1) % NBUF

        @pl.when(c + NBUF - 1 < CPWS)
        def _():
            @pl.when(c > 0)
            def _():
                pltpu.make_async_copy(
                    rows[b2], raw_hbm.at[pl.ds(tok0, CHUNK)],
                    osems[b2]).wait()

            nbase = pl.multiple_of((c + NBUF - 1) * CHUNK, CHUNK)
            pltpu.async_copy(
                word_hbm.at[ids_v.at[pl.ds(nbase, CHUNK)]], rows[b2],
                gsems[b2])

        pltpu.make_async_copy(
            word_hbm.at[pl.ds(0, CHUNK)], rows[b], gsems[b]).wait()
        cbase = pl.multiple_of(c * CHUNK, CHUNK)
        pltpu.async_copy(rows[b], raw_hbm.at[pl.ds(tok0 + cbase, CHUNK)],
                         osems[b])

    def ring(i, _):
        c0 = i * NBUF
        for b in range(NBUF):
            step(c0 + b, b)
        return 0

    lax.fori_loop(0, CPWS // NBUF, ring, 0)

    for b in range(NBUF):
        pltpu.make_async_copy(
            rows[b], raw_hbm.at[pl.ds(tok0, CHUNK)], osems[b]).wait()


def _sc_gather(ids_slice, word_emb):
    mesh = plsc.VectorSubcoreMesh(core_axis_name="c", subcore_axis_name="s")

    def body(ids_hbm, word_hbm, raw_hbm, *scr):
        _sc_body(ids_hbm, word_hbm, raw_hbm, scr[0],
                 list(scr[1:1 + NBUF]), list(scr[1 + NBUF:1 + 2 * NBUF]),
                 list(scr[1 + 2 * NBUF:]))

    f = functools.partial(
        pl.kernel,
        out_type=jax.ShapeDtypeStruct((TSL, HIDDEN), jnp.float32),
        mesh=mesh,
        scratch_types=(
            [pltpu.VMEM((TPWS,), jnp.int32)]
            + [pltpu.VMEM((CHUNK, HIDDEN), jnp.float32)] * NBUF
            + [pltpu.SemaphoreType.DMA] * (2 * NBUF)
        ),
    )(body)
    return f(ids_slice, word_emb)


def _tc_body(raw_ref, tt_ref, pos_ref, typ_ref, gamma_ref, beta_ref,
             out_ref):
    x = raw_ref[...]
    ttf = tt_ref[...].astype(jnp.float32)
    pos = pos_ref[...]
    typ0 = typ_ref[0]
    typd = typ_ref[1] - typ0
    x = x + pos[None] + typ0[None, None] + ttf[..., None] * typd[None, None]
    mean = jnp.mean(x, axis=-1, keepdims=True)
    var = jnp.mean(x * x, axis=-1, keepdims=True) - mean * mean
    y = (x - mean) * lax.rsqrt(var + 1e-12)
    out_ref[...] = y * gamma_ref[...][None, None] + beta_ref[...][None, None]


def _tc_ln(raw_slice, tt_slice, pos_emb, type_emb, gamma, beta):
    grid = (BSL // BB,)
    return pl.pallas_call(
        _tc_body,
        grid=grid,
        in_specs=[
            pl.BlockSpec((BB, L, HIDDEN), lambda i: (i, 0, 0)),
            pl.BlockSpec((BB, L), lambda i: (i, 0)),
            pl.BlockSpec((L, HIDDEN), lambda i: (0, 0)),
            pl.BlockSpec((2, HIDDEN), lambda i: (0, 0)),
            pl.BlockSpec((HIDDEN,), lambda i: (0,)),
            pl.BlockSpec((HIDDEN,), lambda i: (0,)),
        ],
        out_specs=pl.BlockSpec((BB, L, HIDDEN), lambda i: (i, 0, 0)),
        out_shape=jax.ShapeDtypeStruct((BSL, L, HIDDEN), jnp.float32),
    )(raw_slice, tt_slice, pos_emb[:L], type_emb, gamma, beta)


@jax.jit
def _embed(input_ids, token_type_ids, word_emb, pos_emb, type_emb, gamma,
           beta):
    ids1 = input_ids.reshape(B * L)
    raws = [
        _sc_gather(lax.dynamic_slice_in_dim(ids1, s * TSL, TSL), word_emb)
        for s in range(NSLICES)
    ]
    outs = [
        _tc_ln(raws[s].reshape(BSL, L, HIDDEN),
               lax.dynamic_slice_in_dim(token_type_ids, s * BSL, BSL),
               pos_emb, type_emb, gamma, beta)
        for s in range(NSLICES)
    ]
    return jnp.concatenate(outs, axis=0)


def kernel(input_ids, token_type_ids, word_emb, pos_emb, type_emb, gamma,
           beta):
    return _embed(input_ids, token_type_ids, word_emb, pos_emb, type_emb,
                  gamma, beta)
